# baseline (speedup 1.0000x reference)
import functools

import jax
import jax.numpy as jnp
from jax import lax
from jax.experimental import pallas as pl
from jax.experimental.pallas import tpu as pltpu
from jax.experimental.pallas import tpu_sc as plsc

N = 10000
NP = 10240
NPH = NP // 2
H = 32
G = 64
COND = 100
DIMS = [90, 45, 15, 5]
IN_DIMS = [128, 90, 45, 15]
CP = [96, 48, 16, 16]
CPR = [48, 48, 16, 16]
RND = [2, 1, 1, 1]
OW = [48, 48, 32, 32]
E = 320000
ET = E + N
PT = 20736
EP = 16 * PT
KBL = [48, 48, 96, 96]
RPH = NPH // 16
BR = 512
IGN = -1

f32 = jnp.float32
i32 = jnp.int32


def _dense_body(nin, nh, refs):
    ins = refs[:2 * nin]
    w_refs = refs[2 * nin:3 * nin]
    brow_ref = refs[3 * nin]
    outs = refs[3 * nin + 1:]
    y = brow_ref[...]
    for t in range(nin):
        o_ref, bias_ref = ins[2 * t], ins[2 * t + 1]
        xb = o_ref[...] + bias_ref[...]
        xt = jnp.where(xb >= 0, xb, 0.01 * xb)
        y = y + jnp.dot(xt, w_refs[t][...], preferred_element_type=f32)
    hw = outs[0].shape[1]
    for t in range(nh):
        outs[t][...] = y[:, t * hw:(t + 1) * hw]
    outs[nh][...] = jnp.concatenate(
        [y[:, nh * hw:nh * hw + 2 * H],
         jnp.zeros((y.shape[0], 128 - 2 * H), f32)], axis=1)


def _dense1_body(x_ref, w_ref, brow_ref, ha_ref, hb_ref, sd_ref):
    y = jnp.dot(x_ref[...], w_ref[...], preferred_element_type=f32)
    y = y + brow_ref[...]
    hw = ha_ref.shape[1]
    ha_ref[...] = y[:, :hw]
    hb_ref[...] = y[:, hw:2 * hw]
    sd_ref[...] = jnp.concatenate(
        [y[:, 2 * hw:2 * hw + 2 * H],
         jnp.zeros((y.shape[0], 128 - 2 * H), f32)], axis=1)


def _dense_call(layer, args):
    nh = RND[layer]
    cpr = CPR[layer]
    hw = H * cpr
    cols = nh * hw + 2 * H
    out_shape = ([jax.ShapeDtypeStruct((NP, hw), f32)] * nh
                 + [jax.ShapeDtypeStruct((NP, 128), f32)])
    out_specs = ([pl.BlockSpec((BR, hw), lambda i: (i, 0))] * nh
                 + [pl.BlockSpec((BR, 128), lambda i: (i, 0))])
    if layer == 0:
        in_specs = [pl.BlockSpec((BR, 128), lambda i: (i, 0)),
                    pl.BlockSpec((128, cols), lambda i: (0, 0)),
                    pl.BlockSpec((1, cols), lambda i: (0, 0))]
        body = _dense1_body
    else:
        nin = RND[layer - 1]
        cip = OW[layer - 1]
        in_specs = []
        for _ in range(nin):
            in_specs.append(pl.BlockSpec((BR, cip), lambda i: (i, 0)))
            in_specs.append(pl.BlockSpec((1, cip), lambda i: (0, 0)))
        for _ in range(nin):
            in_specs.append(pl.BlockSpec((cip, cols), lambda i: (0, 0)))
        in_specs.append(pl.BlockSpec((1, cols), lambda i: (0, 0)))
        inner = functools.partial(_dense_body, nin, nh)

        def body_fn(*refs):
            inner(refs)

        body = body_fn
    return pl.pallas_call(
        body, grid=(NP // BR,), in_specs=in_specs, out_specs=out_specs,
        out_shape=out_shape)(*args)


@functools.cache
def _make_edge(cpr, ow, kb, rounds):
    hc = H * cpr
    nv = cpr // 16
    mesh = plsc.VectorSubcoreMesh(core_axis_name="c", subcore_axis_name="s",
                                  num_cores=2, num_subcores=16)

    @functools.partial(
        pl.kernel,
        out_type=tuple([jax.ShapeDtypeStruct((NP, 128), f32)]
                       + [jax.ShapeDtypeStruct((NP, ow), f32)] * rounds),
        mesh=mesh,
        scratch_types=[
            pltpu.VMEM((kb,), i32),
            pltpu.VMEM((kb,), i32),
            pltpu.VMEM((kb,), i32),
            pltpu.VMEM((kb,), i32),
            pltpu.VMEM((kb,), i32),
            pltpu.VMEM((kb, 128), f32),
            pltpu.VMEM((kb, 128), f32),
            pltpu.VMEM((kb, ow), f32),
            pltpu.VMEM((kb, 128), f32),
            pltpu.VMEM((16, ow), f32),
            pltpu.VMEM((kb, ow), f32),
            pltpu.VMEM_SHARED((NPH + 16, ow), f32),
            pltpu.SemaphoreType.DMA,
        ] + [pltpu.VMEM((kb,), i32) for _ in range(hc // 128)]
          + [pltpu.VMEM((kb, 128), f32) for _ in range(hc // 128)],
    )
    def edge(*refs):
        src_hbm, dst_hbm, sd_hbm = refs[0], refs[1], refs[2]
        h_tabs = refs[3:3 + rounds]
        zerow_hbm = refs[3 + rounds]
        z_hbm = refs[4 + rounds]
        out_tabs = refs[5 + rounds:5 + 2 * rounds]
        rest = refs[5 + 2 * rounds:]
        (sidx, didx, sidxm, didxg, didxl, sv, dv, ew, zv, zb,
         msg, acc, sems) = rest[:13]
        nt = hc // 128
        hidx = rest[13:13 + nt]
        hbuf = rest[13 + nt:13 + 2 * nt]
        cid = lax.axis_index("c")
        sid = lax.axis_index("s")
        lo = cid * NPH
        r0 = pl.multiple_of(sid * RPH, 64)
        gdst = pl.multiple_of(cid * NPH + sid * RPH, 64)
        pltpu.sync_copy(zerow_hbm.at[pl.ds(r0, RPH)], acc.at[pl.ds(r0, RPH)])
        plsc.subcore_barrier()
        base = sid * PT
        inv_h = 1.0 / H

        dummy = NPH + lax.iota(i32, 16)

        def masks():
            for j in range(kb // 16):
                sl = pl.ds(j * 16, 16)
                d = didx[sl]
                dl = d - lo
                owned = (dl >= 0) & (dl < NPH)
                didxg[sl] = jnp.where(owned, d, 0)
                didxl[sl] = jnp.where(owned, dl, dummy)
                sidxm[sl] = jnp.where(owned, sidx[sl], 0)

        def load_ids(off):
            pltpu.sync_copy(src_hbm.at[pl.ds(off, kb)], sidx)
            pltpu.sync_copy(dst_hbm.at[pl.ds(off, kb)], didx)

        def compute_ew(k):
            for q in range(2):
                a = (sv[k, pl.ds(q * 16, 16)]
                     + dv[k, pl.ds(H + q * 16, 16)])
                a = jnp.where(a >= 0, a, 0.2 * a)
                ew[k, pl.ds(q * 16, 16)] = jnp.exp(a)

        def zero_cols(buf, c_from):
            if c_from >= ow:
                return

            def zrow0(k, c2):
                for q in range(c_from // 16, ow // 16):
                    buf[k, pl.ds(q * 16, 16)] = jnp.zeros((16,), f32)
                return c2

            lax.fori_loop(0, kb, zrow0, 0)

        zero_cols(ew, 32)

        def chunk1(ch, carry):
            off = pl.multiple_of(base + ch * kb, 16)
            load_ids(off)
            masks()
            pltpu.async_copy(
                sd_hbm.at[sidxm], sv,
                sems).wait()
            pltpu.async_copy(
                sd_hbm.at[didxg], dv,
                sems).wait()

            def row(k, c2):
                compute_ew(k)
                return c2

            lax.fori_loop(0, kb, row, 0)
            pltpu.sync_copy(
                ew, acc.at[didxl],
                add=True)
            return carry

        lax.fori_loop(0, PT // kb, chunk1, 0)
        plsc.subcore_barrier()

        def zrow(j, c2):
            ro = pl.multiple_of(r0 + j * 16, 16)
            pltpu.sync_copy(acc.at[pl.ds(ro, 16)], zb)

            def fill(k, c3):
                zv[k, pl.ds(0, 16)] = zb[k, pl.ds(0, 16)]
                zv[k, pl.ds(16, 16)] = zb[k, pl.ds(16, 16)]
                for q in range(2, 8):
                    zv[k, pl.ds(q * 16, 16)] = jnp.zeros((16,), f32)
                return c3

            lax.fori_loop(0, 16, fill, 0)
            pltpu.sync_copy(zv.at[pl.ds(0, 16)],
                            z_hbm.at[pl.ds(gdst + j * 16, 16)])
            return c2

        lax.fori_loop(0, RPH // 16, zrow, 0)
        pltpu.sync_copy(zerow_hbm.at[pl.ds(r0, RPH)], acc.at[pl.ds(r0, RPH)])
        plsc.subcore_barrier()

        zero_cols(msg, nv * 16)

        for rnd in range(rounds):
            h_hbm = h_tabs[rnd]

            def chunk3(ch, carry):
                off = pl.multiple_of(base + ch * kb, 16)
                load_ids(off)
                masks()
                for j in range(kb // 16):
                    sl16 = pl.ds(j * 16, 16)
                    sm = sidxm[sl16] * nt
                    for t in range(nt):
                        hidx[t][sl16] = sm + t
                pltpu.async_copy(
                    sd_hbm.at[sidxm], sv,
                    sems).wait()
                pltpu.async_copy(
                    sd_hbm.at[didxg], dv,
                    sems).wait()
                pltpu.async_copy(
                    z_hbm.at[didxg], zv,
                    sems).wait()

                def roww(k, c2):
                    compute_ew(k)
                    for q in range(2):
                        sl = pl.ds(q * 16, 16)
                        z = zv[k, sl] + 1e-16
                        ew[k, sl] = ew[k, sl] / z * inv_h
                    return c2

                lax.fori_loop(0, kb, roww, 0)
                for t in range(nt):
                    pltpu.async_copy(h_hbm.at[hidx[t]], hbuf[t],
                                     sems).wait()

                def rowm(k, c2):
                    acc_v = [jnp.zeros((16,), f32) for _ in range(nv)]
                    wvecs = [ew[k, pl.ds(0, 16)], ew[k, pl.ds(16, 16)]]
                    for hh in range(H):
                        wv = wvecs[hh // 16][hh % 16]
                        for j in range(nv):
                            c = hh * cpr + j * 16
                            acc_v[j] = acc_v[j] + wv * hbuf[c // 128][
                                k, pl.ds(c % 128, 16)]
                    for j in range(nv):
                        msg[k, pl.ds(j * 16, 16)] = acc_v[j]
                    return c2

                lax.fori_loop(0, kb, rowm, 0)
                pltpu.sync_copy(
                    msg, acc.at[didxl],
                    add=True)
                return carry

            lax.fori_loop(0, PT // kb, chunk3, 0)
            plsc.subcore_barrier()
            pltpu.sync_copy(acc.at[pl.ds(r0, RPH)],
                            out_tabs[rnd].at[pl.ds(gdst, RPH)])
            if rnd + 1 < rounds:
                plsc.subcore_barrier()
                pltpu.sync_copy(zerow_hbm.at[pl.ds(r0, RPH)],
                                acc.at[pl.ds(r0, RPH)])
                plsc.subcore_barrier()

    return edge


def _pool_body(o_ref, bias_ref, b_ref, cond_ref, fcv_ref, fcwc_ref,
               fcb_ref, out_ref):
    xb = o_ref[...] + bias_ref[...]
    x = jnp.where(xb >= 0, xb, 0.01 * xb)
    bcol = b_ref[...]
    gi = lax.broadcasted_iota(i32, (NP, G), 1)
    oneh = (bcol == gi).astype(f32)
    dn = (((0,), (0,)), ((), ()))
    sums = lax.dot_general(oneh, x, dn, preferred_element_type=f32)
    ones = jnp.ones((NP, 1), f32)
    cnt = lax.dot_general(oneh, ones, dn, preferred_element_type=f32)
    num = lax.dot_general(sums, fcv_ref[...], (((1,), (1,)), ((), ())),
                          preferred_element_type=f32)
    c0 = jnp.sum(cond_ref[...] * fcwc_ref[...]) + fcb_ref[0, 0]
    z = num / jnp.maximum(cnt, 1.0) + c0
    out_ref[...] = 1.0 / (1.0 + jnp.exp(-z))


def _pool_call(o, bias4, batch_col, cond_pad, fcv, fcwc, fcb):
    return pl.pallas_call(
        _pool_body,
        out_shape=jax.ShapeDtypeStruct((G, 1), f32),
    )(o, bias4, batch_col, cond_pad, fcv, fcwc, fcb)


def kernel(x, edge_index, batch, conditional_features,
           W1, att_src1, att_dst1, b1, bn_g1, bn_b1,
           W2, att_src2, att_dst2, b2, bn_g2, bn_b2,
           W3, att_src3, att_dst3, b3, bn_g3, bn_b3,
           W4, att_src4, att_dst4, b4, bn_g4, bn_b4,
           fc_w, fc_b):
    params = [
        (W1, att_src1, att_dst1, b1, bn_g1, bn_b1),
        (W2, att_src2, att_dst2, b2, bn_g2, bn_b2),
        (W3, att_src3, att_dst3, b3, bn_g3, bn_b3),
        (W4, att_src4, att_dst4, b4, bn_g4, bn_b4),
    ]
    loops = jnp.arange(N, dtype=i32)
    src = jnp.concatenate([edge_index[0], loops,
                           jnp.zeros((EP - ET,), i32)])
    dst = jnp.concatenate([edge_index[1], loops,
                           jnp.full((EP - ET,), NP - 1, i32)])
    xp = jnp.zeros((NP, 128), f32).at[:N].set(x)
    batch_col = jnp.concatenate(
        [batch, jnp.full((NP - N,), G, i32)]).reshape(NP, 1)
    cond_pad = jnp.zeros((1, 128), f32).at[0, :COND].set(conditional_features)
    zeros_w = {w: jnp.zeros((NP, w), f32) for w in set(OW)}

    cur = None
    for i in range(4):
        W, asrc, adst, b, g, bb = params[i]
        ci, co, cp_, cpr = IN_DIMS[i], DIMS[i], CP[i], CPR[i]
        nh = RND[i]
        Wt = W.reshape(H, co, ci).transpose(2, 0, 1)
        Wp = jnp.pad(Wt, ((0, 0), (0, 0), (0, cp_ - co)))
        hws = [Wp[:, :, t * cpr:(t + 1) * cpr].reshape(ci, H * cpr)
               for t in range(nh)]
        As = jnp.einsum('chd,hd->ch', Wt, asrc)
        Ad = jnp.einsum('chd,hd->ch', Wt, adst)
        Wext = jnp.concatenate(hws + [As, Ad], axis=1)
        brow = (bb @ Wext)[None, :]
        Wext = (g / jnp.sqrt(1.0 + 1e-5))[:, None] * Wext
        if i == 0:
            outs = _dense_call(0, (xp, Wext, brow))
        else:
            nin = RND[i - 1]
            cip = OW[i - 1]
            Wext = jnp.pad(Wext, ((0, nin * cip - ci), (0, 0)))
            bias_full = jnp.pad(params[i - 1][3],
                                (0, nin * cip - DIMS[i - 1]))
            args = []
            for t in range(nin):
                args.append(cur[t])
                args.append(bias_full[t * cip:(t + 1) * cip][None, :])
            for t in range(nin):
                args.append(Wext[t * cip:(t + 1) * cip])
            args.append(brow)
            outs = _dense_call(i, tuple(args))
        h_tabs = [t.reshape(NP * (H * cpr // 128), 128) for t in outs[:nh]]
        sd = outs[nh]
        edge = _make_edge(cpr, OW[i], KBL[i], nh)
        eouts = edge(src, dst, sd, *h_tabs, zeros_w[OW[i]])
        cur = list(eouts[1:])

    bias4 = jnp.pad(b4, (0, OW[3] - DIMS[3]))[None, :]
    fcv = jnp.zeros((1, OW[3]), f32).at[0, :DIMS[3]].set(fc_w[0, :DIMS[3]])
    fcwc = jnp.zeros((1, 128), f32).at[0, :COND].set(fc_w[0, DIMS[3]:])
    fcb = fc_b.reshape(1, 1)
    return _pool_call(cur[0], bias4, batch_col, cond_pad, fcv, fcwc, fcb)


# speedup vs baseline: 1.0074x; 1.0074x over previous
import functools

import jax
import jax.numpy as jnp
from jax import lax
from jax.experimental import pallas as pl
from jax.experimental.pallas import tpu as pltpu
from jax.experimental.pallas import tpu_sc as plsc

N = 10000
NP = 10240
NPH = NP // 2
H = 32
G = 64
COND = 100
DIMS = [90, 45, 15, 5]
IN_DIMS = [128, 90, 45, 15]
CP = [96, 48, 16, 16]
CPR = [48, 48, 16, 16]
RND = [2, 1, 1, 1]
OW = [48, 48, 32, 32]
E = 320000
ET = E + N
PT = 20736
EP = 16 * PT
KBL = [48, 48, 96, 96]
RPH = NPH // 16
BR = 512
IGN = -1

f32 = jnp.float32
i32 = jnp.int32


def _dense_body(nin, nh, refs):
    ins = refs[:2 * nin]
    w_refs = refs[2 * nin:3 * nin]
    brow_ref = refs[3 * nin]
    outs = refs[3 * nin + 1:]
    y = brow_ref[...]
    for t in range(nin):
        o_ref, bias_ref = ins[2 * t], ins[2 * t + 1]
        xb = o_ref[...] + bias_ref[...]
        xt = jnp.where(xb >= 0, xb, 0.01 * xb)
        y = y + jnp.dot(xt, w_refs[t][...], preferred_element_type=f32)
    hw = outs[0].shape[1]
    for t in range(nh):
        outs[t][...] = y[:, t * hw:(t + 1) * hw]
    outs[nh][...] = jnp.concatenate(
        [y[:, nh * hw:nh * hw + 2 * H],
         jnp.zeros((y.shape[0], 128 - 2 * H), f32)], axis=1)


def _dense1_body(x_ref, w_ref, brow_ref, ha_ref, hb_ref, sd_ref):
    y = jnp.dot(x_ref[...], w_ref[...], preferred_element_type=f32)
    y = y + brow_ref[...]
    hw = ha_ref.shape[1]
    ha_ref[...] = y[:, :hw]
    hb_ref[...] = y[:, hw:2 * hw]
    sd_ref[...] = jnp.concatenate(
        [y[:, 2 * hw:2 * hw + 2 * H],
         jnp.zeros((y.shape[0], 128 - 2 * H), f32)], axis=1)


def _dense_call(layer, args):
    nh = RND[layer]
    cpr = CPR[layer]
    hw = H * cpr
    cols = nh * hw + 2 * H
    out_shape = ([jax.ShapeDtypeStruct((NP, hw), f32)] * nh
                 + [jax.ShapeDtypeStruct((NP, 128), f32)])
    out_specs = ([pl.BlockSpec((BR, hw), lambda i: (i, 0))] * nh
                 + [pl.BlockSpec((BR, 128), lambda i: (i, 0))])
    if layer == 0:
        in_specs = [pl.BlockSpec((BR, 128), lambda i: (i, 0)),
                    pl.BlockSpec((128, cols), lambda i: (0, 0)),
                    pl.BlockSpec((1, cols), lambda i: (0, 0))]
        body = _dense1_body
    else:
        nin = RND[layer - 1]
        cip = OW[layer - 1]
        in_specs = []
        for _ in range(nin):
            in_specs.append(pl.BlockSpec((BR, cip), lambda i: (i, 0)))
            in_specs.append(pl.BlockSpec((1, cip), lambda i: (0, 0)))
        for _ in range(nin):
            in_specs.append(pl.BlockSpec((cip, cols), lambda i: (0, 0)))
        in_specs.append(pl.BlockSpec((1, cols), lambda i: (0, 0)))
        inner = functools.partial(_dense_body, nin, nh)

        def body_fn(*refs):
            inner(refs)

        body = body_fn
    return pl.pallas_call(
        body, grid=(NP // BR,), in_specs=in_specs, out_specs=out_specs,
        out_shape=out_shape)(*args)


@functools.cache
def _make_edge(cpr, ow, kb, rounds):
    hc = H * cpr
    nv = cpr // 16
    mesh = plsc.VectorSubcoreMesh(core_axis_name="c", subcore_axis_name="s",
                                  num_cores=2, num_subcores=16)

    @functools.partial(
        pl.kernel,
        out_type=tuple([jax.ShapeDtypeStruct((NP, 128), f32)]
                       + [jax.ShapeDtypeStruct((NP, ow), f32)] * rounds),
        mesh=mesh,
        scratch_types=[
            pltpu.VMEM((kb,), i32),
            pltpu.VMEM((kb,), i32),
            pltpu.VMEM((kb,), i32),
            pltpu.VMEM((kb,), i32),
            pltpu.VMEM((kb,), i32),
            pltpu.VMEM((kb, 128), f32),
            pltpu.VMEM((kb, 128), f32),
            pltpu.VMEM((kb, ow), f32),
            pltpu.VMEM((kb, 128), f32),
            pltpu.VMEM((16, ow), f32),
            pltpu.VMEM((kb, ow), f32),
            pltpu.VMEM_SHARED((NPH + 16, ow), f32),
            pltpu.SemaphoreType.DMA,
        ] + [pltpu.VMEM((kb,), i32) for _ in range(hc // 128)]
          + [pltpu.VMEM((kb, 128), f32) for _ in range(hc // 128)],
    )
    def edge(*refs):
        src_hbm, dst_hbm, sd_hbm = refs[0], refs[1], refs[2]
        h_tabs = refs[3:3 + rounds]
        zerow_hbm = refs[3 + rounds]
        z_hbm = refs[4 + rounds]
        out_tabs = refs[5 + rounds:5 + 2 * rounds]
        rest = refs[5 + 2 * rounds:]
        (sidx, didx, sidxm, didxg, didxl, sv, dv, ew, zv, zb,
         msg, acc, sems) = rest[:13]
        nt = hc // 128
        hidx = rest[13:13 + nt]
        hbuf = rest[13 + nt:13 + 2 * nt]
        cid = lax.axis_index("c")
        sid = lax.axis_index("s")
        lo = cid * NPH
        r0 = pl.multiple_of(sid * RPH, 64)
        gdst = pl.multiple_of(cid * NPH + sid * RPH, 64)
        pltpu.sync_copy(zerow_hbm.at[pl.ds(r0, RPH)], acc.at[pl.ds(r0, RPH)])
        plsc.subcore_barrier()
        base = sid * PT
        inv_h = 1.0 / H

        dummy = NPH + lax.iota(i32, 16)

        def masks():
            for j in range(kb // 16):
                sl = pl.ds(j * 16, 16)
                d = didx[sl]
                dl = d - lo
                owned = (dl >= 0) & (dl < NPH)
                didxg[sl] = jnp.where(owned, d, 0)
                didxl[sl] = jnp.where(owned, dl, dummy)
                sidxm[sl] = jnp.where(owned, sidx[sl], 0)

        def load_ids(off):
            pltpu.sync_copy(src_hbm.at[pl.ds(off, kb)], sidx)
            pltpu.sync_copy(dst_hbm.at[pl.ds(off, kb)], didx)

        def compute_ew(k):
            for q in range(2):
                a = (sv[k, pl.ds(q * 16, 16)]
                     + dv[k, pl.ds(H + q * 16, 16)])
                a = jnp.where(a >= 0, a, 0.2 * a)
                ew[k, pl.ds(q * 16, 16)] = jnp.exp(a)

        def zero_cols(buf, c_from):
            if c_from >= ow:
                return

            def zrow0(k, c2):
                for q in range(c_from // 16, ow // 16):
                    buf[k, pl.ds(q * 16, 16)] = jnp.zeros((16,), f32)
                return c2

            lax.fori_loop(0, kb, zrow0, 0)

        zero_cols(ew, 32)

        def chunk1(ch, carry):
            off = pl.multiple_of(base + ch * kb, 16)
            load_ids(off)
            masks()
            d1 = pltpu.async_copy(sd_hbm.at[sidxm], sv, sems)
            d2 = pltpu.async_copy(sd_hbm.at[didxg], dv, sems)
            d1.wait()
            d2.wait()

            def row(k, c2):
                compute_ew(k)
                return c2

            lax.fori_loop(0, kb, row, 0)
            pltpu.sync_copy(
                ew, acc.at[didxl],
                add=True)
            return carry

        lax.fori_loop(0, PT // kb, chunk1, 0)
        plsc.subcore_barrier()

        def zrow(j, c2):
            ro = pl.multiple_of(r0 + j * 16, 16)
            pltpu.sync_copy(acc.at[pl.ds(ro, 16)], zb)

            def fill(k, c3):
                zv[k, pl.ds(0, 16)] = zb[k, pl.ds(0, 16)]
                zv[k, pl.ds(16, 16)] = zb[k, pl.ds(16, 16)]
                for q in range(2, 8):
                    zv[k, pl.ds(q * 16, 16)] = jnp.zeros((16,), f32)
                return c3

            lax.fori_loop(0, 16, fill, 0)
            pltpu.sync_copy(zv.at[pl.ds(0, 16)],
                            z_hbm.at[pl.ds(gdst + j * 16, 16)])
            return c2

        lax.fori_loop(0, RPH // 16, zrow, 0)
        pltpu.sync_copy(zerow_hbm.at[pl.ds(r0, RPH)], acc.at[pl.ds(r0, RPH)])
        plsc.subcore_barrier()

        zero_cols(msg, nv * 16)

        for rnd in range(rounds):
            h_hbm = h_tabs[rnd]

            def chunk3(ch, carry):
                off = pl.multiple_of(base + ch * kb, 16)
                load_ids(off)
                masks()
                for j in range(kb // 16):
                    sl16 = pl.ds(j * 16, 16)
                    sm = sidxm[sl16] * nt
                    for t in range(nt):
                        hidx[t][sl16] = sm + t
                descs = [
                    pltpu.async_copy(sd_hbm.at[sidxm], sv, sems),
                    pltpu.async_copy(sd_hbm.at[didxg], dv, sems),
                    pltpu.async_copy(z_hbm.at[didxg], zv, sems),
                ]
                descs += [pltpu.async_copy(h_hbm.at[hidx[t]], hbuf[t], sems)
                          for t in range(nt)]
                for dd in descs:
                    dd.wait()

                def roww(k, c2):
                    compute_ew(k)
                    for q in range(2):
                        sl = pl.ds(q * 16, 16)
                        z = zv[k, sl] + 1e-16
                        ew[k, sl] = ew[k, sl] / z * inv_h
                    return c2

                lax.fori_loop(0, kb, roww, 0)

                def rowm(k, c2):
                    acc_v = [jnp.zeros((16,), f32) for _ in range(nv)]
                    wvecs = [ew[k, pl.ds(0, 16)], ew[k, pl.ds(16, 16)]]
                    for hh in range(H):
                        wv = wvecs[hh // 16][hh % 16]
                        for j in range(nv):
                            c = hh * cpr + j * 16
                            acc_v[j] = acc_v[j] + wv * hbuf[c // 128][
                                k, pl.ds(c % 128, 16)]
                    for j in range(nv):
                        msg[k, pl.ds(j * 16, 16)] = acc_v[j]
                    return c2

                lax.fori_loop(0, kb, rowm, 0)
                pltpu.sync_copy(
                    msg, acc.at[didxl],
                    add=True)
                return carry

            lax.fori_loop(0, PT // kb, chunk3, 0)
            plsc.subcore_barrier()
            pltpu.sync_copy(acc.at[pl.ds(r0, RPH)],
                            out_tabs[rnd].at[pl.ds(gdst, RPH)])
            if rnd + 1 < rounds:
                plsc.subcore_barrier()
                pltpu.sync_copy(zerow_hbm.at[pl.ds(r0, RPH)],
                                acc.at[pl.ds(r0, RPH)])
                plsc.subcore_barrier()

    return edge


def _pool_body(o_ref, bias_ref, b_ref, cond_ref, fcv_ref, fcwc_ref,
               fcb_ref, out_ref):
    xb = o_ref[...] + bias_ref[...]
    x = jnp.where(xb >= 0, xb, 0.01 * xb)
    bcol = b_ref[...]
    gi = lax.broadcasted_iota(i32, (NP, G), 1)
    oneh = (bcol == gi).astype(f32)
    dn = (((0,), (0,)), ((), ()))
    sums = lax.dot_general(oneh, x, dn, preferred_element_type=f32)
    ones = jnp.ones((NP, 1), f32)
    cnt = lax.dot_general(oneh, ones, dn, preferred_element_type=f32)
    num = lax.dot_general(sums, fcv_ref[...], (((1,), (1,)), ((), ())),
                          preferred_element_type=f32)
    c0 = jnp.sum(cond_ref[...] * fcwc_ref[...]) + fcb_ref[0, 0]
    z = num / jnp.maximum(cnt, 1.0) + c0
    out_ref[...] = 1.0 / (1.0 + jnp.exp(-z))


def _pool_call(o, bias4, batch_col, cond_pad, fcv, fcwc, fcb):
    return pl.pallas_call(
        _pool_body,
        out_shape=jax.ShapeDtypeStruct((G, 1), f32),
    )(o, bias4, batch_col, cond_pad, fcv, fcwc, fcb)


def kernel(x, edge_index, batch, conditional_features,
           W1, att_src1, att_dst1, b1, bn_g1, bn_b1,
           W2, att_src2, att_dst2, b2, bn_g2, bn_b2,
           W3, att_src3, att_dst3, b3, bn_g3, bn_b3,
           W4, att_src4, att_dst4, b4, bn_g4, bn_b4,
           fc_w, fc_b):
    params = [
        (W1, att_src1, att_dst1, b1, bn_g1, bn_b1),
        (W2, att_src2, att_dst2, b2, bn_g2, bn_b2),
        (W3, att_src3, att_dst3, b3, bn_g3, bn_b3),
        (W4, att_src4, att_dst4, b4, bn_g4, bn_b4),
    ]
    loops = jnp.arange(N, dtype=i32)
    src = jnp.concatenate([edge_index[0], loops,
                           jnp.zeros((EP - ET,), i32)])
    dst = jnp.concatenate([edge_index[1], loops,
                           jnp.full((EP - ET,), NP - 1, i32)])
    xp = jnp.zeros((NP, 128), f32).at[:N].set(x)
    batch_col = jnp.concatenate(
        [batch, jnp.full((NP - N,), G, i32)]).reshape(NP, 1)
    cond_pad = jnp.zeros((1, 128), f32).at[0, :COND].set(conditional_features)
    zeros_w = {w: jnp.zeros((NP, w), f32) for w in set(OW)}

    cur = None
    for i in range(4):
        W, asrc, adst, b, g, bb = params[i]
        ci, co, cp_, cpr = IN_DIMS[i], DIMS[i], CP[i], CPR[i]
        nh = RND[i]
        Wt = W.reshape(H, co, ci).transpose(2, 0, 1)
        Wp = jnp.pad(Wt, ((0, 0), (0, 0), (0, cp_ - co)))
        hws = [Wp[:, :, t * cpr:(t + 1) * cpr].reshape(ci, H * cpr)
               for t in range(nh)]
        As = jnp.einsum('chd,hd->ch', Wt, asrc)
        Ad = jnp.einsum('chd,hd->ch', Wt, adst)
        Wext = jnp.concatenate(hws + [As, Ad], axis=1)
        brow = (bb @ Wext)[None, :]
        Wext = (g / jnp.sqrt(1.0 + 1e-5))[:, None] * Wext
        if i == 0:
            outs = _dense_call(0, (xp, Wext, brow))
        else:
            nin = RND[i - 1]
            cip = OW[i - 1]
            Wext = jnp.pad(Wext, ((0, nin * cip - ci), (0, 0)))
            bias_full = jnp.pad(params[i - 1][3],
                                (0, nin * cip - DIMS[i - 1]))
            args = []
            for t in range(nin):
                args.append(cur[t])
                args.append(bias_full[t * cip:(t + 1) * cip][None, :])
            for t in range(nin):
                args.append(Wext[t * cip:(t + 1) * cip])
            args.append(brow)
            outs = _dense_call(i, tuple(args))
        h_tabs = [t.reshape(NP * (H * cpr // 128), 128) for t in outs[:nh]]
        sd = outs[nh]
        edge = _make_edge(cpr, OW[i], KBL[i], nh)
        eouts = edge(src, dst, sd, *h_tabs, zeros_w[OW[i]])
        cur = list(eouts[1:])

    bias4 = jnp.pad(b4, (0, OW[3] - DIMS[3]))[None, :]
    fcv = jnp.zeros((1, OW[3]), f32).at[0, :DIMS[3]].set(fc_w[0, :DIMS[3]])
    fcwc = jnp.zeros((1, 128), f32).at[0, :COND].set(fc_w[0, DIMS[3]:])
    fcb = fc_b.reshape(1, 1)
    return _pool_call(cur[0], bias4, batch_col, cond_pad, fcv, fcwc, fcb)


# speedup vs baseline: 1.0075x; 1.0002x over previous
import functools

import jax
import jax.numpy as jnp
from jax import lax
from jax.experimental import pallas as pl
from jax.experimental.pallas import tpu as pltpu
from jax.experimental.pallas import tpu_sc as plsc

N = 10000
NP = 10240
NPH = NP // 2
H = 32
G = 64
COND = 100
DIMS = [90, 45, 15, 5]
IN_DIMS = [128, 90, 45, 15]
CP = [96, 48, 16, 16]
CPR = [48, 48, 16, 16]
RND = [2, 1, 1, 1]
OW = [48, 48, 32, 32]
E = 320000
ET = E + N
PT = 20736
EP = 16 * PT
KBL = [48, 48, 96, 96]
RPH = NPH // 16
BR = 512
IGN = -1

f32 = jnp.float32
i32 = jnp.int32


def _dense_body(nin, nh, refs):
    ins = refs[:2 * nin]
    w_refs = refs[2 * nin:3 * nin]
    brow_ref = refs[3 * nin]
    outs = refs[3 * nin + 1:]
    y = brow_ref[...]
    for t in range(nin):
        o_ref, bias_ref = ins[2 * t], ins[2 * t + 1]
        xb = o_ref[...] + bias_ref[...]
        xt = jnp.where(xb >= 0, xb, 0.01 * xb)
        y = y + jnp.dot(xt, w_refs[t][...], preferred_element_type=f32)
    hw = outs[0].shape[1]
    for t in range(nh):
        outs[t][...] = y[:, t * hw:(t + 1) * hw]
    outs[nh][...] = jnp.concatenate(
        [y[:, nh * hw:nh * hw + 2 * H],
         jnp.zeros((y.shape[0], 128 - 2 * H), f32)], axis=1)


def _dense1_body(x_ref, w_ref, brow_ref, ha_ref, hb_ref, sd_ref):
    y = jnp.dot(x_ref[...], w_ref[...], preferred_element_type=f32)
    y = y + brow_ref[...]
    hw = ha_ref.shape[1]
    ha_ref[...] = y[:, :hw]
    hb_ref[...] = y[:, hw:2 * hw]
    sd_ref[...] = jnp.concatenate(
        [y[:, 2 * hw:2 * hw + 2 * H],
         jnp.zeros((y.shape[0], 128 - 2 * H), f32)], axis=1)


def _dense_call(layer, args):
    nh = RND[layer]
    cpr = CPR[layer]
    hw = H * cpr
    cols = nh * hw + 2 * H
    out_shape = ([jax.ShapeDtypeStruct((NP, hw), f32)] * nh
                 + [jax.ShapeDtypeStruct((NP, 128), f32)])
    out_specs = ([pl.BlockSpec((BR, hw), lambda i: (i, 0))] * nh
                 + [pl.BlockSpec((BR, 128), lambda i: (i, 0))])
    if layer == 0:
        in_specs = [pl.BlockSpec((BR, 128), lambda i: (i, 0)),
                    pl.BlockSpec((128, cols), lambda i: (0, 0)),
                    pl.BlockSpec((1, cols), lambda i: (0, 0))]
        body = _dense1_body
    else:
        nin = RND[layer - 1]
        cip = OW[layer - 1]
        in_specs = []
        for _ in range(nin):
            in_specs.append(pl.BlockSpec((BR, cip), lambda i: (i, 0)))
            in_specs.append(pl.BlockSpec((1, cip), lambda i: (0, 0)))
        for _ in range(nin):
            in_specs.append(pl.BlockSpec((cip, cols), lambda i: (0, 0)))
        in_specs.append(pl.BlockSpec((1, cols), lambda i: (0, 0)))
        inner = functools.partial(_dense_body, nin, nh)

        def body_fn(*refs):
            inner(refs)

        body = body_fn
    return pl.pallas_call(
        body, grid=(NP // BR,), in_specs=in_specs, out_specs=out_specs,
        out_shape=out_shape)(*args)


@functools.cache
def _make_edge(cpr, ow, kb, rounds):
    hc = H * cpr
    nv = cpr // 16
    mesh = plsc.VectorSubcoreMesh(core_axis_name="c", subcore_axis_name="s",
                                  num_cores=2, num_subcores=16)

    @functools.partial(
        pl.kernel,
        out_type=tuple([jax.ShapeDtypeStruct((NP, 128), f32)]
                       + [jax.ShapeDtypeStruct((NP, ow), f32)] * rounds),
        mesh=mesh,
        scratch_types=[
            pltpu.VMEM((kb,), i32),
            pltpu.VMEM((kb,), i32),
            pltpu.VMEM((kb,), i32),
            pltpu.VMEM((kb,), i32),
            pltpu.VMEM((kb,), i32),
            pltpu.VMEM((kb, 128), f32),
            pltpu.VMEM((kb, 128), f32),
            pltpu.VMEM((kb, ow), f32),
            pltpu.VMEM((kb, 128), f32),
            pltpu.VMEM((16, ow), f32),
            pltpu.VMEM((kb, ow), f32),
            pltpu.VMEM_SHARED((NPH + 16, ow), f32),
            pltpu.SemaphoreType.DMA,
        ] + [pltpu.VMEM((kb,), i32) for _ in range(hc // 128)]
          + [pltpu.VMEM((kb, 128), f32) for _ in range(hc // 128)],
    )
    def edge(*refs):
        src_hbm, dst_hbm, sd_hbm = refs[0], refs[1], refs[2]
        h_tabs = refs[3:3 + rounds]
        zerow_hbm = refs[3 + rounds]
        z_hbm = refs[4 + rounds]
        out_tabs = refs[5 + rounds:5 + 2 * rounds]
        rest = refs[5 + 2 * rounds:]
        (sidx, didx, sidxm, didxg, didxl, sv, dv, ew, zv, zb,
         msg, acc, sems) = rest[:13]
        nt = hc // 128
        hidx = rest[13:13 + nt]
        hbuf = rest[13 + nt:13 + 2 * nt]
        cid = lax.axis_index("c")
        sid = lax.axis_index("s")
        lo = cid * NPH
        r0 = pl.multiple_of(sid * RPH, 64)
        gdst = pl.multiple_of(cid * NPH + sid * RPH, 64)
        pltpu.sync_copy(zerow_hbm.at[pl.ds(r0, RPH)], acc.at[pl.ds(r0, RPH)])
        plsc.subcore_barrier()
        base = sid * PT
        inv_h = 1.0 / H

        dummy = NPH + lax.iota(i32, 16)

        def masks():
            for j in range(kb // 16):
                sl = pl.ds(j * 16, 16)
                d = didx[sl]
                dl = d - lo
                owned = (dl >= 0) & (dl < NPH)
                didxg[sl] = jnp.where(owned, d, 0)
                didxl[sl] = jnp.where(owned, dl, dummy)
                sidxm[sl] = jnp.where(owned, sidx[sl], 0)

        def load_ids(off):
            pltpu.sync_copy(src_hbm.at[pl.ds(off, kb)], sidx)
            pltpu.sync_copy(dst_hbm.at[pl.ds(off, kb)], didx)

        def compute_ew(k):
            for q in range(2):
                a = (sv[k, pl.ds(q * 16, 16)]
                     + dv[k, pl.ds(H + q * 16, 16)])
                a = jnp.where(a >= 0, a, 0.2 * a)
                ew[k, pl.ds(q * 16, 16)] = jnp.exp(a)

        def zero_cols(buf, c_from):
            if c_from >= ow:
                return

            def zrow0(k, c2):
                for q in range(c_from // 16, ow // 16):
                    buf[k, pl.ds(q * 16, 16)] = jnp.zeros((16,), f32)
                return c2

            lax.fori_loop(0, kb, zrow0, 0)

        zero_cols(ew, 32)

        def chunk1(ch, carry):
            off = pl.multiple_of(base + ch * kb, 16)
            load_ids(off)
            masks()
            d1 = pltpu.async_copy(sd_hbm.at[sidxm], sv, sems)
            d2 = pltpu.async_copy(sd_hbm.at[didxg], dv, sems)
            d1.wait()
            d2.wait()

            def row(u, c2):
                compute_ew(u * 2)
                compute_ew(u * 2 + 1)
                return c2

            lax.fori_loop(0, kb // 2, row, 0)
            pltpu.sync_copy(
                ew, acc.at[didxl],
                add=True)
            return carry

        lax.fori_loop(0, PT // kb, chunk1, 0)
        plsc.subcore_barrier()

        def zrow(j, c2):
            ro = pl.multiple_of(r0 + j * 16, 16)
            pltpu.sync_copy(acc.at[pl.ds(ro, 16)], zb)

            def fill(k, c3):
                zv[k, pl.ds(0, 16)] = zb[k, pl.ds(0, 16)]
                zv[k, pl.ds(16, 16)] = zb[k, pl.ds(16, 16)]
                for q in range(2, 8):
                    zv[k, pl.ds(q * 16, 16)] = jnp.zeros((16,), f32)
                return c3

            lax.fori_loop(0, 16, fill, 0)
            pltpu.sync_copy(zv.at[pl.ds(0, 16)],
                            z_hbm.at[pl.ds(gdst + j * 16, 16)])
            return c2

        lax.fori_loop(0, RPH // 16, zrow, 0)
        pltpu.sync_copy(zerow_hbm.at[pl.ds(r0, RPH)], acc.at[pl.ds(r0, RPH)])
        plsc.subcore_barrier()

        zero_cols(msg, nv * 16)

        for rnd in range(rounds):
            h_hbm = h_tabs[rnd]

            def chunk3(ch, carry):
                off = pl.multiple_of(base + ch * kb, 16)
                load_ids(off)
                masks()
                for j in range(kb // 16):
                    sl16 = pl.ds(j * 16, 16)
                    sm = sidxm[sl16] * nt
                    for t in range(nt):
                        hidx[t][sl16] = sm + t
                descs = [
                    pltpu.async_copy(sd_hbm.at[sidxm], sv, sems),
                    pltpu.async_copy(sd_hbm.at[didxg], dv, sems),
                    pltpu.async_copy(z_hbm.at[didxg], zv, sems),
                ]
                descs += [pltpu.async_copy(h_hbm.at[hidx[t]], hbuf[t], sems)
                          for t in range(nt)]
                for dd in descs:
                    dd.wait()

                def roww(u, c2):
                    for k in (u * 2, u * 2 + 1):
                        compute_ew(k)
                        for q in range(2):
                            sl = pl.ds(q * 16, 16)
                            z = zv[k, sl] + 1e-16
                            ew[k, sl] = ew[k, sl] / z * inv_h
                    return c2

                lax.fori_loop(0, kb // 2, roww, 0)

                def rowm(u, c2):
                    ks = [u * 2, u * 2 + 1]
                    acc_v = [[jnp.zeros((16,), f32) for _ in range(nv)]
                             for _ in ks]
                    wvecs = [[ew[k, pl.ds(0, 16)], ew[k, pl.ds(16, 16)]]
                             for k in ks]
                    for hh in range(H):
                        for e in range(2):
                            wv = wvecs[e][hh // 16][hh % 16]
                            for j in range(nv):
                                c = hh * cpr + j * 16
                                acc_v[e][j] = acc_v[e][j] + wv * hbuf[
                                    c // 128][ks[e], pl.ds(c % 128, 16)]
                    for e in range(2):
                        for j in range(nv):
                            msg[ks[e], pl.ds(j * 16, 16)] = acc_v[e][j]
                    return c2

                lax.fori_loop(0, kb // 2, rowm, 0)
                pltpu.sync_copy(
                    msg, acc.at[didxl],
                    add=True)
                return carry

            lax.fori_loop(0, PT // kb, chunk3, 0)
            plsc.subcore_barrier()
            pltpu.sync_copy(acc.at[pl.ds(r0, RPH)],
                            out_tabs[rnd].at[pl.ds(gdst, RPH)])
            if rnd + 1 < rounds:
                plsc.subcore_barrier()
                pltpu.sync_copy(zerow_hbm.at[pl.ds(r0, RPH)],
                                acc.at[pl.ds(r0, RPH)])
                plsc.subcore_barrier()

    return edge


def _pool_body(o_ref, bias_ref, b_ref, cond_ref, fcv_ref, fcwc_ref,
               fcb_ref, out_ref):
    xb = o_ref[...] + bias_ref[...]
    x = jnp.where(xb >= 0, xb, 0.01 * xb)
    bcol = b_ref[...]
    gi = lax.broadcasted_iota(i32, (NP, G), 1)
    oneh = (bcol == gi).astype(f32)
    dn = (((0,), (0,)), ((), ()))
    sums = lax.dot_general(oneh, x, dn, preferred_element_type=f32)
    ones = jnp.ones((NP, 1), f32)
    cnt = lax.dot_general(oneh, ones, dn, preferred_element_type=f32)
    num = lax.dot_general(sums, fcv_ref[...], (((1,), (1,)), ((), ())),
                          preferred_element_type=f32)
    c0 = jnp.sum(cond_ref[...] * fcwc_ref[...]) + fcb_ref[0, 0]
    z = num / jnp.maximum(cnt, 1.0) + c0
    out_ref[...] = 1.0 / (1.0 + jnp.exp(-z))


def _pool_call(o, bias4, batch_col, cond_pad, fcv, fcwc, fcb):
    return pl.pallas_call(
        _pool_body,
        out_shape=jax.ShapeDtypeStruct((G, 1), f32),
    )(o, bias4, batch_col, cond_pad, fcv, fcwc, fcb)


def kernel(x, edge_index, batch, conditional_features,
           W1, att_src1, att_dst1, b1, bn_g1, bn_b1,
           W2, att_src2, att_dst2, b2, bn_g2, bn_b2,
           W3, att_src3, att_dst3, b3, bn_g3, bn_b3,
           W4, att_src4, att_dst4, b4, bn_g4, bn_b4,
           fc_w, fc_b):
    params = [
        (W1, att_src1, att_dst1, b1, bn_g1, bn_b1),
        (W2, att_src2, att_dst2, b2, bn_g2, bn_b2),
        (W3, att_src3, att_dst3, b3, bn_g3, bn_b3),
        (W4, att_src4, att_dst4, b4, bn_g4, bn_b4),
    ]
    loops = jnp.arange(N, dtype=i32)
    src = jnp.concatenate([edge_index[0], loops,
                           jnp.zeros((EP - ET,), i32)])
    dst = jnp.concatenate([edge_index[1], loops,
                           jnp.full((EP - ET,), NP - 1, i32)])
    xp = jnp.zeros((NP, 128), f32).at[:N].set(x)
    batch_col = jnp.concatenate(
        [batch, jnp.full((NP - N,), G, i32)]).reshape(NP, 1)
    cond_pad = jnp.zeros((1, 128), f32).at[0, :COND].set(conditional_features)
    zeros_w = {w: jnp.zeros((NP, w), f32) for w in set(OW)}

    cur = None
    for i in range(4):
        W, asrc, adst, b, g, bb = params[i]
        ci, co, cp_, cpr = IN_DIMS[i], DIMS[i], CP[i], CPR[i]
        nh = RND[i]
        Wt = W.reshape(H, co, ci).transpose(2, 0, 1)
        Wp = jnp.pad(Wt, ((0, 0), (0, 0), (0, cp_ - co)))
        hws = [Wp[:, :, t * cpr:(t + 1) * cpr].reshape(ci, H * cpr)
               for t in range(nh)]
        As = jnp.einsum('chd,hd->ch', Wt, asrc)
        Ad = jnp.einsum('chd,hd->ch', Wt, adst)
        Wext = jnp.concatenate(hws + [As, Ad], axis=1)
        brow = (bb @ Wext)[None, :]
        Wext = (g / jnp.sqrt(1.0 + 1e-5))[:, None] * Wext
        if i == 0:
            outs = _dense_call(0, (xp, Wext, brow))
        else:
            nin = RND[i - 1]
            cip = OW[i - 1]
            Wext = jnp.pad(Wext, ((0, nin * cip - ci), (0, 0)))
            bias_full = jnp.pad(params[i - 1][3],
                                (0, nin * cip - DIMS[i - 1]))
            args = []
            for t in range(nin):
                args.append(cur[t])
                args.append(bias_full[t * cip:(t + 1) * cip][None, :])
            for t in range(nin):
                args.append(Wext[t * cip:(t + 1) * cip])
            args.append(brow)
            outs = _dense_call(i, tuple(args))
        h_tabs = [t.reshape(NP * (H * cpr // 128), 128) for t in outs[:nh]]
        sd = outs[nh]
        edge = _make_edge(cpr, OW[i], KBL[i], nh)
        eouts = edge(src, dst, sd, *h_tabs, zeros_w[OW[i]])
        cur = list(eouts[1:])

    bias4 = jnp.pad(b4, (0, OW[3] - DIMS[3]))[None, :]
    fcv = jnp.zeros((1, OW[3]), f32).at[0, :DIMS[3]].set(fc_w[0, :DIMS[3]])
    fcwc = jnp.zeros((1, 128), f32).at[0, :COND].set(fc_w[0, DIMS[3]:])
    fcb = fc_b.reshape(1, 1)
    return _pool_call(cur[0], bias4, batch_col, cond_pad, fcv, fcwc, fcb)
